# baseline (speedup 1.0000x reference)
import jax
import jax.numpy as jnp
from jax.experimental import pallas as pl


def _fused_kernel(a_ref, b_ref, x_ref, ao_ref, bo_ref):
    a = a_ref[...]
    b = b_ref[...]
    ao_ref[...] = a
    bo_ref[...] = b
    x_ref[...] = jnp.sum(a * b, axis=1, keepdims=True)


def kernel(gum, gim):
    n, d = gum.shape
    x2d, a_out, b_out = pl.pallas_call(
        _fused_kernel,
        out_shape=(
            jax.ShapeDtypeStruct((n, 1), jnp.float32),
            jax.ShapeDtypeStruct((n, d), jnp.float32),
            jax.ShapeDtypeStruct((n, d), jnp.float32),
        ),
    )(gum, gim)
    return (x2d.reshape(n), a_out, b_out)
